# baseline (speedup 1.0000x reference)
import functools

import jax
import jax.numpy as jnp
from jax import lax
from jax.experimental import pallas as pl
from jax.experimental.pallas import tpu as pltpu
from jax.experimental.pallas import tpu_sc as plsc

H = 4
K = 8
DIM_KEY = 64
NUM_KEYS = 128
NUM_EXPERTS = 16384
HID = 1024
T = 2048

NC = 2
NS = 16
NW = NC * NS
TPW = T // NW
NSEL = H * K
L = 16
NCH = HID // L
NCHUNKS = 2


TT = 512


def _routing_body(x_ref, wq_ref, keys_ref, idx_ref, score_ref):
    x = x_ref[...]
    iota_nk = lax.broadcasted_iota(jnp.int32, (NUM_KEYS, TT), 0)
    iota_kk = lax.broadcasted_iota(jnp.int32, (K * K, TT), 0)
    neg_inf = jnp.float32(-jnp.inf)
    idx_rows = []
    score_rows = []
    for h in range(H):
        tops = []
        for s in range(2):
            row0 = (2 * h + s) * DIM_KEY
            wq = wq_ref[row0:row0 + DIM_KEY, :]
            q = lax.dot_general(
                x, wq, (((1,), (1,)), ((), ())),
                preferred_element_type=jnp.float32,
                precision=lax.Precision.DEFAULT,
            )
            kt = keys_ref[h, s]
            simT = lax.dot_general(
                kt, q.T, (((1,), (0,)), ((), ())),
                preferred_element_type=jnp.float32,
                precision=lax.Precision.DEFAULT,
            )
            svals = []
            sidx = []
            cur = simT
            for _ in range(K):
                m = jnp.max(cur, axis=0, keepdims=True)
                pos = jnp.min(
                    jnp.where(cur == m, iota_nk, NUM_KEYS), axis=0, keepdims=True
                )
                svals.append(m)
                sidx.append(pos)
                cur = jnp.where(iota_nk == pos, neg_inf, cur)
            tops.append((svals, sidx))
        (s1, i1), (s2, i2) = tops
        s2_full = jnp.concatenate(s2, axis=0)
        i2_full = jnp.concatenate(i2, axis=0)
        cand_s = jnp.concatenate([s1[i] + s2_full for i in range(K)], axis=0)
        cand_i = jnp.concatenate(
            [i1[i] * NUM_KEYS + i2_full for i in range(K)], axis=0
        )
        for _ in range(K):
            m = jnp.max(cand_s, axis=0, keepdims=True)
            pos = jnp.min(
                jnp.where(cand_s == m, iota_kk, K * K), axis=0, keepdims=True
            )
            sel = iota_kk == pos
            e = jnp.sum(jnp.where(sel, cand_i, 0), axis=0, keepdims=True)
            idx_rows.append(e)
            score_rows.append(m)
            cand_s = jnp.where(sel, neg_inf, cand_s)
    idx_ref[...] = jnp.concatenate(idx_rows, axis=0)
    score_ref[...] = jnp.concatenate(score_rows, axis=0)


@functools.lru_cache(maxsize=None)
def _build_routing(tokens):
    return pl.pallas_call(
        _routing_body,
        grid=(tokens // TT,),
        in_specs=[
            pl.BlockSpec((TT, HID), lambda i: (i, 0)),
            pl.BlockSpec((DIM_KEY * H * 2, HID), lambda i: (0, 0)),
            pl.BlockSpec((H, 2, NUM_KEYS, DIM_KEY), lambda i: (0, 0, 0, 0)),
        ],
        out_specs=(
            pl.BlockSpec((NSEL, TT), lambda i: (0, i)),
            pl.BlockSpec((NSEL, TT), lambda i: (0, i)),
        ),
        out_shape=(
            jax.ShapeDtypeStruct((NSEL, tokens), jnp.int32),
            jax.ShapeDtypeStruct((NSEL, tokens), jnp.float32),
        ),
    )


def _sc_body(tpw, x_hbm, idx_hbm, score_hbm, down_hbm, up_hbm, out_hbm,
             idx_all, sc_all, x_a, x_b, d_a, d_b, up_v, h_v, out_a, out_b,
             sem_da, sem_db, sem_up, sem_xa, sem_xb, sem_oa, sem_ob):
    wid = lax.axis_index("s") * NC + lax.axis_index("c")
    base = wid * tpw
    iota16 = lax.broadcasted_iota(jnp.int32, (L,), 0)

    def compute(t, x_v, down_v, out_v):
        for g in range(2):
            def c_body(c, accs):
                out = accs
                for u in range(2):
                    sl = pl.ds((2 * c + u) * L, L)
                    xc = x_v[sl]
                    out = tuple(
                        out[j] + xc * down_v[g * 16 + j, sl] for j in range(16)
                    )
                return out
            accs = lax.fori_loop(
                0, NCH // 2, c_body,
                tuple(jnp.zeros((L,), jnp.float32) for _ in range(16)),
            )
            hacc = jnp.zeros((L,), jnp.float32)
            for j in range(16):
                hacc = jnp.where(iota16 == j, jnp.sum(accs[j]), hacc)
            sc = sc_all[t, pl.ds(g * L, L)]
            w = hacc * (1.0 / (1.0 + jnp.exp(-hacc))) * jnp.maximum(sc, 0.0)
            h_v[pl.ds(g * L, L)] = w

    def accum(up_ref, out_v):
        hv0 = h_v[pl.ds(0, L)]
        hv1 = h_v[pl.ds(L, L)]
        ws = tuple(hv0[j] for j in range(L)) + tuple(hv1[j] for j in range(L))

        def c_body2(c, carry2):
            for u in range(2):
                sl = pl.ds((2 * c + u) * L, L)
                acc = ws[0] * up_ref[0, sl]
                for j in range(1, NSEL):
                    acc = acc + ws[j] * up_ref[j, sl]
                out_v[sl] = acc
            return carry2

        lax.fori_loop(0, NCH // 2, c_body2, 0)

    pltpu.sync_copy(idx_hbm.at[pl.ds(base, tpw)], idx_all)
    pltpu.sync_copy(score_hbm.at[pl.ds(base, tpw)], sc_all)
    pltpu.sync_copy(x_hbm.at[base], x_a)
    pltpu.async_copy(down_hbm.at[idx_all.at[0]], d_a, sem_da)
    pltpu.async_copy(up_hbm.at[idx_all.at[0]], up_v, sem_up)
    cp_xb0 = pltpu.async_copy(x_hbm.at[base + 1], x_b, sem_xb)

    def iter_body(i, carry):
        a = 2 * i
        b = 2 * i + 1
        pltpu.make_async_copy(down_hbm.at[idx_all.at[a]], d_a, sem_da).wait()

        @pl.when(i > 0)
        def _():
            pltpu.make_async_copy(x_hbm.at[base + a], x_a, sem_xa).wait()

        compute(a, x_a, d_a, out_a)

        @pl.when(i < tpw // 2 - 1)
        def _():
            pltpu.async_copy(x_hbm.at[base + a + 2], x_a, sem_xa)

        pltpu.make_async_copy(up_hbm.at[idx_all.at[a]], up_v, sem_up).wait()
        pltpu.async_copy(down_hbm.at[idx_all.at[a + 1]], d_b, sem_db)

        @pl.when(i > 0)
        def _():
            pltpu.make_async_copy(out_a, out_hbm.at[base + a - 2], sem_oa).wait()

        accum(up_v, out_a)
        pltpu.async_copy(up_hbm.at[idx_all.at[a + 1]], up_v, sem_up)
        pltpu.async_copy(out_a, out_hbm.at[base + a], sem_oa)

        pltpu.make_async_copy(down_hbm.at[idx_all.at[b]], d_b, sem_db).wait()
        pltpu.make_async_copy(x_hbm.at[base + b], x_b, sem_xb).wait()
        compute(b, x_b, d_b, out_b)

        @pl.when(i < tpw // 2 - 1)
        def _():
            pltpu.async_copy(x_hbm.at[base + b + 2], x_b, sem_xb)

        pltpu.make_async_copy(up_hbm.at[idx_all.at[b]], up_v, sem_up).wait()

        @pl.when(i < tpw // 2 - 1)
        def _():
            pltpu.async_copy(down_hbm.at[idx_all.at[b + 1]], d_a, sem_da)

        @pl.when(i > 0)
        def _():
            pltpu.make_async_copy(out_b, out_hbm.at[base + b - 2], sem_ob).wait()

        accum(up_v, out_b)

        @pl.when(i < tpw // 2 - 1)
        def _():
            pltpu.async_copy(up_hbm.at[idx_all.at[b + 1]], up_v, sem_up)

        pltpu.async_copy(out_b, out_hbm.at[base + b], sem_ob)
        return carry

    lax.fori_loop(0, tpw // 2, iter_body, 0)
    pltpu.make_async_copy(out_a, out_hbm.at[base + tpw - 2], sem_oa).wait()
    pltpu.make_async_copy(out_b, out_hbm.at[base + tpw - 1], sem_ob).wait()


@functools.lru_cache(maxsize=None)
def _build_sc_combine(tokens):
    tpw = tokens // NW
    return pl.kernel(
        functools.partial(_sc_body, tpw),
        out_type=jax.ShapeDtypeStruct((tokens, HID), jnp.float32),
        mesh=plsc.VectorSubcoreMesh(
            core_axis_name="c", subcore_axis_name="s",
            num_cores=NC, num_subcores=NS,
        ),
        compiler_params=pltpu.CompilerParams(needs_layout_passes=False),
        scratch_types=[
            pltpu.VMEM((tpw, NSEL), jnp.int32),
            pltpu.VMEM((tpw, NSEL), jnp.float32),
            pltpu.VMEM((HID,), jnp.float32),
            pltpu.VMEM((HID,), jnp.float32),
            pltpu.VMEM((NSEL, HID), jnp.float32),
            pltpu.VMEM((NSEL, HID), jnp.float32),
            pltpu.VMEM((NSEL, HID), jnp.float32),
            pltpu.VMEM((NSEL,), jnp.float32),
            pltpu.VMEM((HID,), jnp.float32),
            pltpu.VMEM((HID,), jnp.float32),
            pltpu.SemaphoreType.DMA,
            pltpu.SemaphoreType.DMA,
            pltpu.SemaphoreType.DMA,
            pltpu.SemaphoreType.DMA,
            pltpu.SemaphoreType.DMA,
            pltpu.SemaphoreType.DMA,
            pltpu.SemaphoreType.DMA,
        ],
    )


def kernel(x, W_q, keys, expert_down, expert_up):
    B, T_, D = x.shape
    xf = x.reshape(T_, D)
    keys_r = keys.transpose(0, 2, 1, 3)
    tok = T_ // NCHUNKS
    outs = []
    for c in range(NCHUNKS):
        xc = xf[c * tok:(c + 1) * tok]
        idx_t, scores_t = _build_routing(tok)(xc, W_q, keys_r)
        outs.append(
            _build_sc_combine(tok)(xc, idx_t.T, scores_t.T,
                                   expert_down, expert_up))
    out = jnp.concatenate(outs, axis=0)
    return out.reshape(B, T_, D)


# speedup vs baseline: 1.0208x; 1.0208x over previous
import functools

import jax
import jax.numpy as jnp
from jax import lax
from jax.experimental import pallas as pl
from jax.experimental.pallas import tpu as pltpu
from jax.experimental.pallas import tpu_sc as plsc

H = 4
K = 8
DIM_KEY = 64
NUM_KEYS = 128
NUM_EXPERTS = 16384
HID = 1024
T = 2048

NC = 2
NS = 16
NW = NC * NS
TPW = T // NW
NSEL = H * K
L = 16
NCH = HID // L
NCHUNKS = 1


TT = 1024


def _routing_body(x_ref, wq_ref, keys_ref, idx_ref, score_ref):
    x = x_ref[...]
    iota_nk = lax.broadcasted_iota(jnp.int32, (NUM_KEYS, TT), 0)
    iota_kk = lax.broadcasted_iota(jnp.int32, (K * K, TT), 0)
    neg_inf = jnp.float32(-jnp.inf)
    idx_rows = []
    score_rows = []
    for h in range(H):
        tops = []
        for s in range(2):
            row0 = (2 * h + s) * DIM_KEY
            wq = wq_ref[row0:row0 + DIM_KEY, :]
            q = lax.dot_general(
                x, wq, (((1,), (1,)), ((), ())),
                preferred_element_type=jnp.float32,
                precision=lax.Precision.DEFAULT,
            )
            kt = keys_ref[h, s]
            simT = lax.dot_general(
                kt, q.T, (((1,), (0,)), ((), ())),
                preferred_element_type=jnp.float32,
                precision=lax.Precision.DEFAULT,
            )
            svals = []
            sidx = []
            cur = simT
            for _ in range(K):
                m = jnp.max(cur, axis=0, keepdims=True)
                pos = jnp.min(
                    jnp.where(cur == m, iota_nk, NUM_KEYS), axis=0, keepdims=True
                )
                svals.append(m)
                sidx.append(pos)
                cur = jnp.where(iota_nk == pos, neg_inf, cur)
            tops.append((svals, sidx))
        (s1, i1), (s2, i2) = tops
        s2_full = jnp.concatenate(s2, axis=0)
        i2_full = jnp.concatenate(i2, axis=0)
        cand_s = jnp.concatenate([s1[i] + s2_full for i in range(K)], axis=0)
        cand_i = jnp.concatenate(
            [i1[i] * NUM_KEYS + i2_full for i in range(K)], axis=0
        )
        for _ in range(K):
            m = jnp.max(cand_s, axis=0, keepdims=True)
            pos = jnp.min(
                jnp.where(cand_s == m, iota_kk, K * K), axis=0, keepdims=True
            )
            sel = iota_kk == pos
            e = jnp.sum(jnp.where(sel, cand_i, 0), axis=0, keepdims=True)
            idx_rows.append(e)
            score_rows.append(m)
            cand_s = jnp.where(sel, neg_inf, cand_s)
    idx_ref[...] = jnp.concatenate(idx_rows, axis=0)
    score_ref[...] = jnp.concatenate(score_rows, axis=0)


@functools.lru_cache(maxsize=None)
def _build_routing(tokens):
    return pl.pallas_call(
        _routing_body,
        grid=(tokens // TT,),
        in_specs=[
            pl.BlockSpec((TT, HID), lambda i: (i, 0)),
            pl.BlockSpec((DIM_KEY * H * 2, HID), lambda i: (0, 0)),
            pl.BlockSpec((H, 2, NUM_KEYS, DIM_KEY), lambda i: (0, 0, 0, 0)),
        ],
        out_specs=(
            pl.BlockSpec((NSEL, TT), lambda i: (0, i)),
            pl.BlockSpec((NSEL, TT), lambda i: (0, i)),
        ),
        out_shape=(
            jax.ShapeDtypeStruct((NSEL, tokens), jnp.int32),
            jax.ShapeDtypeStruct((NSEL, tokens), jnp.float32),
        ),
    )


def _sc_body(tpw, x_hbm, idx_hbm, score_hbm, down_hbm, up_hbm, out_hbm,
             idx_all, sc_all, x_a, x_b, d_a, d_b, up_v, h_v, out_a, out_b,
             sem_da, sem_db, sem_up, sem_xa, sem_xb, sem_oa, sem_ob):
    wid = lax.axis_index("s") * NC + lax.axis_index("c")
    base = wid * tpw
    iota16 = lax.broadcasted_iota(jnp.int32, (L,), 0)

    def compute(t, x_v, down_v, out_v):
        for g in range(2):
            def c_body(c, accs):
                out = accs
                for u in range(2):
                    sl = pl.ds((2 * c + u) * L, L)
                    xc = x_v[sl]
                    out = tuple(
                        out[j] + xc * down_v[g * 16 + j, sl] for j in range(16)
                    )
                return out
            accs = lax.fori_loop(
                0, NCH // 2, c_body,
                tuple(jnp.zeros((L,), jnp.float32) for _ in range(16)),
            )
            hacc = jnp.zeros((L,), jnp.float32)
            for j in range(16):
                hacc = jnp.where(iota16 == j, jnp.sum(accs[j]), hacc)
            sc = sc_all[t, pl.ds(g * L, L)]
            w = hacc * (1.0 / (1.0 + jnp.exp(-hacc))) * jnp.maximum(sc, 0.0)
            h_v[pl.ds(g * L, L)] = w

    def accum(up_ref, out_v):
        hv0 = h_v[pl.ds(0, L)]
        hv1 = h_v[pl.ds(L, L)]
        ws = tuple(hv0[j] for j in range(L)) + tuple(hv1[j] for j in range(L))

        def c_body2(c, carry2):
            for u in range(2):
                sl = pl.ds((2 * c + u) * L, L)
                acc = ws[0] * up_ref[0, sl]
                for j in range(1, NSEL):
                    acc = acc + ws[j] * up_ref[j, sl]
                out_v[sl] = acc
            return carry2

        lax.fori_loop(0, NCH // 2, c_body2, 0)

    pltpu.sync_copy(idx_hbm.at[pl.ds(base, tpw)], idx_all)
    pltpu.sync_copy(score_hbm.at[pl.ds(base, tpw)], sc_all)
    pltpu.sync_copy(x_hbm.at[base], x_a)
    pltpu.async_copy(down_hbm.at[idx_all.at[0]], d_a, sem_da)
    pltpu.async_copy(up_hbm.at[idx_all.at[0]], up_v, sem_up)
    cp_xb0 = pltpu.async_copy(x_hbm.at[base + 1], x_b, sem_xb)

    def iter_body(i, carry):
        a = 2 * i
        b = 2 * i + 1
        pltpu.make_async_copy(down_hbm.at[idx_all.at[a]], d_a, sem_da).wait()

        @pl.when(i > 0)
        def _():
            pltpu.make_async_copy(x_hbm.at[base + a], x_a, sem_xa).wait()

        compute(a, x_a, d_a, out_a)

        @pl.when(i < tpw // 2 - 1)
        def _():
            pltpu.async_copy(x_hbm.at[base + a + 2], x_a, sem_xa)

        pltpu.make_async_copy(up_hbm.at[idx_all.at[a]], up_v, sem_up).wait()
        pltpu.async_copy(down_hbm.at[idx_all.at[a + 1]], d_b, sem_db)

        @pl.when(i > 0)
        def _():
            pltpu.make_async_copy(out_a, out_hbm.at[base + a - 2], sem_oa).wait()

        accum(up_v, out_a)
        pltpu.async_copy(up_hbm.at[idx_all.at[a + 1]], up_v, sem_up)
        pltpu.async_copy(out_a, out_hbm.at[base + a], sem_oa)

        pltpu.make_async_copy(down_hbm.at[idx_all.at[b]], d_b, sem_db).wait()
        pltpu.make_async_copy(x_hbm.at[base + b], x_b, sem_xb).wait()
        compute(b, x_b, d_b, out_b)

        @pl.when(i < tpw // 2 - 1)
        def _():
            pltpu.async_copy(x_hbm.at[base + b + 2], x_b, sem_xb)

        pltpu.make_async_copy(up_hbm.at[idx_all.at[b]], up_v, sem_up).wait()

        @pl.when(i < tpw // 2 - 1)
        def _():
            pltpu.async_copy(down_hbm.at[idx_all.at[b + 1]], d_a, sem_da)

        @pl.when(i > 0)
        def _():
            pltpu.make_async_copy(out_b, out_hbm.at[base + b - 2], sem_ob).wait()

        accum(up_v, out_b)

        @pl.when(i < tpw // 2 - 1)
        def _():
            pltpu.async_copy(up_hbm.at[idx_all.at[b + 1]], up_v, sem_up)

        pltpu.async_copy(out_b, out_hbm.at[base + b], sem_ob)
        return carry

    lax.fori_loop(0, tpw // 2, iter_body, 0)
    pltpu.make_async_copy(out_a, out_hbm.at[base + tpw - 2], sem_oa).wait()
    pltpu.make_async_copy(out_b, out_hbm.at[base + tpw - 1], sem_ob).wait()


@functools.lru_cache(maxsize=None)
def _build_sc_combine(tokens):
    tpw = tokens // NW
    return pl.kernel(
        functools.partial(_sc_body, tpw),
        out_type=jax.ShapeDtypeStruct((tokens, HID), jnp.float32),
        mesh=plsc.VectorSubcoreMesh(
            core_axis_name="c", subcore_axis_name="s",
            num_cores=NC, num_subcores=NS,
        ),
        compiler_params=pltpu.CompilerParams(needs_layout_passes=False),
        scratch_types=[
            pltpu.VMEM((tpw, NSEL), jnp.int32),
            pltpu.VMEM((tpw, NSEL), jnp.float32),
            pltpu.VMEM((HID,), jnp.float32),
            pltpu.VMEM((HID,), jnp.float32),
            pltpu.VMEM((NSEL, HID), jnp.float32),
            pltpu.VMEM((NSEL, HID), jnp.float32),
            pltpu.VMEM((NSEL, HID), jnp.float32),
            pltpu.VMEM((NSEL,), jnp.float32),
            pltpu.VMEM((HID,), jnp.float32),
            pltpu.VMEM((HID,), jnp.float32),
            pltpu.SemaphoreType.DMA,
            pltpu.SemaphoreType.DMA,
            pltpu.SemaphoreType.DMA,
            pltpu.SemaphoreType.DMA,
            pltpu.SemaphoreType.DMA,
            pltpu.SemaphoreType.DMA,
            pltpu.SemaphoreType.DMA,
        ],
    )


def kernel(x, W_q, keys, expert_down, expert_up):
    B, T_, D = x.shape
    xf = x.reshape(T_, D)
    keys_r = keys.transpose(0, 2, 1, 3)
    tok = T_ // NCHUNKS
    outs = []
    for c in range(NCHUNKS):
        xc = xf[c * tok:(c + 1) * tok]
        idx_t, scores_t = _build_routing(tok)(xc, W_q, keys_r)
        outs.append(
            _build_sc_combine(tok)(xc, idx_t.T, scores_t.T,
                                   expert_down, expert_up))
    out = jnp.concatenate(outs, axis=0)
    return out.reshape(B, T_, D)


# speedup vs baseline: 1.0226x; 1.0017x over previous
import functools

import jax
import jax.numpy as jnp
from jax import lax
from jax.experimental import pallas as pl
from jax.experimental.pallas import tpu as pltpu
from jax.experimental.pallas import tpu_sc as plsc

H = 4
K = 8
DIM_KEY = 64
NUM_KEYS = 128
NUM_EXPERTS = 16384
HID = 1024
T = 2048

NC = 2
NS = 16
NW = NC * NS
NSEL = H * K
L = 16
NCH = HID // L
NCHUNKS = 1


TT = 1024


def _routing_body(x_ref, wq_ref, keys_ref, idx_ref, score_ref):
    x = x_ref[...]
    iota_nk = lax.broadcasted_iota(jnp.int32, (NUM_KEYS, TT), 0)
    iota_kk = lax.broadcasted_iota(jnp.int32, (K * K, TT), 0)
    neg_inf = jnp.float32(-jnp.inf)
    idx_rows = []
    score_rows = []
    for h in range(H):
        tops = []
        for s in range(2):
            row0 = (2 * h + s) * DIM_KEY
            wq = wq_ref[row0:row0 + DIM_KEY, :]
            q = lax.dot_general(
                x, wq, (((1,), (1,)), ((), ())),
                preferred_element_type=jnp.float32,
                precision=lax.Precision.DEFAULT,
            )
            kt = keys_ref[h, s]
            simT = lax.dot_general(
                kt, q.T, (((1,), (0,)), ((), ())),
                preferred_element_type=jnp.float32,
                precision=lax.Precision.DEFAULT,
            )
            svals = []
            sidx = []
            cur = simT
            for _ in range(K):
                m = jnp.max(cur, axis=0, keepdims=True)
                pos = jnp.min(
                    jnp.where(cur == m, iota_nk, NUM_KEYS), axis=0, keepdims=True
                )
                svals.append(m)
                sidx.append(pos)
                cur = jnp.where(iota_nk == pos, neg_inf, cur)
            tops.append((svals, sidx))
        (s1, i1), (s2, i2) = tops
        s2_full = jnp.concatenate(s2, axis=0)
        i2_full = jnp.concatenate(i2, axis=0)
        cand_s = jnp.concatenate([s1[i] + s2_full for i in range(K)], axis=0)
        cand_i = jnp.concatenate(
            [i1[i] * NUM_KEYS + i2_full for i in range(K)], axis=0
        )
        for _ in range(K):
            m = jnp.max(cand_s, axis=0, keepdims=True)
            pos = jnp.min(
                jnp.where(cand_s == m, iota_kk, K * K), axis=0, keepdims=True
            )
            sel = iota_kk == pos
            e = jnp.sum(jnp.where(sel, cand_i, 0), axis=0, keepdims=True)
            idx_rows.append(e)
            score_rows.append(m)
            cand_s = jnp.where(sel, neg_inf, cand_s)
    idx_ref[...] = jnp.concatenate(idx_rows, axis=0)
    score_ref[...] = jnp.concatenate(score_rows, axis=0)


@functools.lru_cache(maxsize=None)
def _build_routing(tokens):
    return pl.pallas_call(
        _routing_body,
        grid=(tokens // TT,),
        in_specs=[
            pl.BlockSpec((TT, HID), lambda i: (i, 0)),
            pl.BlockSpec((DIM_KEY * H * 2, HID), lambda i: (0, 0)),
            pl.BlockSpec((H, 2, NUM_KEYS, DIM_KEY), lambda i: (0, 0, 0, 0)),
        ],
        out_specs=(
            pl.BlockSpec((NSEL, TT), lambda i: (0, i)),
            pl.BlockSpec((NSEL, TT), lambda i: (0, i)),
        ),
        out_shape=(
            jax.ShapeDtypeStruct((NSEL, tokens), jnp.int32),
            jax.ShapeDtypeStruct((NSEL, tokens), jnp.float32),
        ),
    )


def _sc_body(tpw, x_hbm, idx_hbm, score_hbm, down_hbm, up_hbm, out_hbm,
             idx_all, sc_all, x_a, x_b, d_a, d_b, up_v, h_v, out_a, out_b,
             sem_da, sem_db, sem_up, sem_xa, sem_xb, sem_oa, sem_ob):
    wid = lax.axis_index("s") * NC + lax.axis_index("c")
    base = wid * tpw
    iota16 = lax.broadcasted_iota(jnp.int32, (L,), 0)

    def compute(t, x_v, down_v, out_v):
        for g in range(2):
            def c_body(c, accs):
                out = accs
                for u in range(2):
                    sl = pl.ds((2 * c + u) * L, L)
                    xc = x_v[sl]
                    out = tuple(
                        out[j] + xc * down_v[g * 16 + j, sl] for j in range(16)
                    )
                return out
            accs = lax.fori_loop(
                0, NCH // 2, c_body,
                tuple(jnp.zeros((L,), jnp.float32) for _ in range(16)),
            )
            hacc = jnp.zeros((L,), jnp.float32)
            for j in range(16):
                hacc = jnp.where(iota16 == j, jnp.sum(accs[j]), hacc)
            sc = sc_all[t, pl.ds(g * L, L)]
            w = hacc * (1.0 / (1.0 + jnp.exp(-hacc))) * jnp.maximum(sc, 0.0)
            h_v[pl.ds(g * L, L)] = w

    def accum(up_ref, out_v):
        hv0 = h_v[pl.ds(0, L)]
        hv1 = h_v[pl.ds(L, L)]
        ws = tuple(hv0[j] for j in range(L)) + tuple(hv1[j] for j in range(L))

        def c_body2(c, carry2):
            for u in range(2):
                sl = pl.ds((2 * c + u) * L, L)
                acc = ws[0] * up_ref[0, sl]
                for j in range(1, NSEL):
                    acc = acc + ws[j] * up_ref[j, sl]
                out_v[sl] = acc
            return carry2

        lax.fori_loop(0, NCH // 2, c_body2, 0)

    pltpu.sync_copy(idx_hbm.at[pl.ds(base, tpw)], idx_all)
    pltpu.sync_copy(score_hbm.at[pl.ds(base, tpw)], sc_all)
    pltpu.sync_copy(x_hbm.at[base], x_a)
    pltpu.async_copy(down_hbm.at[idx_all.at[0]], d_a, sem_da)
    pltpu.async_copy(up_hbm.at[idx_all.at[0]], up_v, sem_up)
    pltpu.async_copy(x_hbm.at[base + 1], x_b, sem_xb)

    def iter_body(i, carry):
        a = 2 * i
        b = 2 * i + 1
        pltpu.make_async_copy(down_hbm.at[idx_all.at[a]], d_a, sem_da).wait()

        @pl.when(i > 0)
        def _():
            pltpu.make_async_copy(x_hbm.at[base + a], x_a, sem_xa).wait()

        compute(a, x_a, d_a, out_a)

        @pl.when(i < tpw // 2 - 1)
        def _():
            pltpu.async_copy(x_hbm.at[base + a + 2], x_a, sem_xa)

        pltpu.make_async_copy(up_hbm.at[idx_all.at[a]], up_v, sem_up).wait()
        pltpu.async_copy(down_hbm.at[idx_all.at[a + 1]], d_b, sem_db)

        @pl.when(i > 0)
        def _():
            pltpu.make_async_copy(out_a, out_hbm.at[base + a - 2], sem_oa).wait()

        accum(up_v, out_a)
        pltpu.async_copy(up_hbm.at[idx_all.at[a + 1]], up_v, sem_up)
        pltpu.async_copy(out_a, out_hbm.at[base + a], sem_oa)

        pltpu.make_async_copy(down_hbm.at[idx_all.at[b]], d_b, sem_db).wait()
        pltpu.make_async_copy(x_hbm.at[base + b], x_b, sem_xb).wait()
        compute(b, x_b, d_b, out_b)

        @pl.when(i < tpw // 2 - 1)
        def _():
            pltpu.async_copy(x_hbm.at[base + b + 2], x_b, sem_xb)

        pltpu.make_async_copy(up_hbm.at[idx_all.at[b]], up_v, sem_up).wait()

        @pl.when(i < tpw // 2 - 1)
        def _():
            pltpu.async_copy(down_hbm.at[idx_all.at[b + 1]], d_a, sem_da)

        @pl.when(i > 0)
        def _():
            pltpu.make_async_copy(out_b, out_hbm.at[base + b - 2], sem_ob).wait()

        accum(up_v, out_b)

        @pl.when(i < tpw // 2 - 1)
        def _():
            pltpu.async_copy(up_hbm.at[idx_all.at[b + 1]], up_v, sem_up)

        pltpu.async_copy(out_b, out_hbm.at[base + b], sem_ob)
        return carry

    lax.fori_loop(0, tpw // 2, iter_body, 0)
    pltpu.make_async_copy(out_a, out_hbm.at[base + tpw - 2], sem_oa).wait()
    pltpu.make_async_copy(out_b, out_hbm.at[base + tpw - 1], sem_ob).wait()


@functools.lru_cache(maxsize=None)
def _build_sc_combine(tokens):
    tpw = tokens // NW
    return pl.kernel(
        functools.partial(_sc_body, tpw),
        out_type=jax.ShapeDtypeStruct((tokens, HID), jnp.float32),
        mesh=plsc.VectorSubcoreMesh(
            core_axis_name="c", subcore_axis_name="s",
            num_cores=NC, num_subcores=NS,
        ),
        compiler_params=pltpu.CompilerParams(needs_layout_passes=False),
        scratch_types=[
            pltpu.VMEM((tpw, NSEL), jnp.int32),
            pltpu.VMEM((tpw, NSEL), jnp.float32),
            pltpu.VMEM((HID,), jnp.float32),
            pltpu.VMEM((HID,), jnp.float32),
            pltpu.VMEM((NSEL, HID), jnp.float32),
            pltpu.VMEM((NSEL, HID), jnp.float32),
            pltpu.VMEM((NSEL, HID), jnp.float32),
            pltpu.VMEM((NSEL,), jnp.float32),
            pltpu.VMEM((HID,), jnp.float32),
            pltpu.VMEM((HID,), jnp.float32),
            pltpu.SemaphoreType.DMA,
            pltpu.SemaphoreType.DMA,
            pltpu.SemaphoreType.DMA,
            pltpu.SemaphoreType.DMA,
            pltpu.SemaphoreType.DMA,
            pltpu.SemaphoreType.DMA,
            pltpu.SemaphoreType.DMA,
        ],
    )


def kernel(x, W_q, keys, expert_down, expert_up):
    B, T_, D = x.shape
    xf = x.reshape(T_, D)
    keys_r = keys.transpose(0, 2, 1, 3)
    tok = T_ // NCHUNKS
    outs = []
    for c in range(NCHUNKS):
        xc = xf[c * tok:(c + 1) * tok]
        idx_t, scores_t = _build_routing(tok)(xc, W_q, keys_r)
        outs.append(
            _build_sc_combine(tok)(xc, idx_t.T, scores_t.T,
                                   expert_down, expert_up))
    out = jnp.concatenate(outs, axis=0)
    return out.reshape(B, T_, D)
